# baseline (speedup 1.0000x reference)
import functools

import jax
import jax.numpy as jnp
from jax import lax
from jax.experimental import pallas as pl
from jax.experimental.pallas import tpu as pltpu
from jax.experimental.pallas import tpu_sc as plsc

B = 8
T = 8192
H = 1024
V = 100000

TOK = T // B
NW = 32
RPH = 4
WPB = NW // RPH
VT = 784
NT = VT // WPB
VPW = NT * 128
NVREG = TOK // 16


def _token_weights_body(h_ref, w_ref, b_ref, o_ref):
    x = h_ref[...]
    w = w_ref[...]
    p3 = (x * w).reshape(o_ref.shape[0], 128, H)
    s = jnp.sum(p3, axis=2)
    o_ref[...] = jnp.maximum(s + b_ref[...], 0.0)


def _token_weights(hidden_states, W, b, half):
    blk = 2048
    return pl.pallas_call(
        _token_weights_body,
        grid=(T // 2 // blk,),
        in_specs=[
            pl.BlockSpec((blk, H), lambda i: (half * 2 + i, 0)),
            pl.BlockSpec((1, H), lambda i: (0, 0)),
            pl.BlockSpec((1, 1), lambda i: (0, 0)),
        ],
        out_specs=pl.BlockSpec((blk // 128, 128), lambda i: (i, 0)),
        out_shape=jax.ShapeDtypeStruct((T // 2 // 128, 128), jnp.float32),
    )(hidden_states, W, b.reshape(1, 1))


def _make_scatter_half(row_base):
    @functools.partial(
        pl.kernel,
        mesh=plsc.VectorSubcoreMesh(core_axis_name="c", subcore_axis_name="s"),
        out_type=jax.ShapeDtypeStruct((VT, RPH, 128), jnp.float32),
        compiler_params=pltpu.CompilerParams(
            needs_layout_passes=False, use_tc_tiling_on_sc=False),
        scratch_types=[
            pltpu.VMEM((TOK,), jnp.int32),
            pltpu.VMEM((TOK // 128, 128), jnp.float32),
            pltpu.VMEM((NT, 128), jnp.float32),
            pltpu.SemaphoreType.DMA,
            pltpu.SemaphoreType.DMA,
        ],
    )
    def _scatter_half(ids_hbm, tw_hbm, out_hbm, ids_v, tw_v, acc_v, sem_i, sem_w):
        wid = lax.axis_index("s") * 2 + lax.axis_index("c")
        sub = wid // WPB
        q = wid % WPB
        v0 = q * VPW

        cp_i = pltpu.async_copy(
            ids_hbm.at[pl.ds((row_base + sub) * TOK, TOK)], ids_v, sem_i)
        cp_w = pltpu.async_copy(
            tw_hbm.at[pl.ds(sub * (TOK // 128), TOK // 128), :], tw_v, sem_w)

        zeros16 = jnp.zeros((16,), jnp.float32)

        def zero_body(i, carry):
            for j in range(16):
                acc_v[i * 2 + j // 8, pl.ds((j % 8) * 16, 16)] = zeros16
            return carry

        lax.fori_loop(0, NT // 2, zero_body, 0)
        cp_i.wait()
        cp_w.wait()

        iota = jnp.arange(16, dtype=jnp.int32)

        def perm(x, idx):
            return jnp.take_along_axis(x, idx, axis=0)

        def one_vec(k):
            ids16 = ids_v[pl.ds(k * 16, 16)]
            w16 = tw_v[k // 8, pl.ds((k % 8) * 16, 16)]
            addr = ids16 - v0
            inr = (addr >= 0) & (addr < VPW)
            key = jnp.where(inr, addr, VPW)
            sk, sv = plsc.sort_key_val(key, w16)
            for s in (1, 2, 4, 8):
                up = jnp.minimum(iota + s, 15)
                k_sh = perm(sk, up)
                v_sh = perm(sv, up)
                sv = jnp.where(k_sh == sk, jnp.maximum(sv, v_sh), sv)
            k_dn = perm(sk, jnp.maximum(iota - 1, 0))
            head = (iota == 0) | (sk != k_dn)
            valid = head & (sk < VPW)
            skc = jnp.where(valid, sk, 0)
            t16 = skc // 128
            c16 = skc % 128
            cur = plsc.load_gather(acc_v, [t16, c16], mask=valid)
            plsc.store_scatter(acc_v, [t16, c16], jnp.maximum(cur, sv), mask=valid)

        def vec_body(i, carry):
            for j in range(2):
                one_vec(i * 2 + j)
            return carry

        lax.fori_loop(0, NVREG // 2, vec_body, 0)

        pltpu.sync_copy(acc_v, out_hbm.at[pl.ds(q * NT, NT), sub, :])

    return _scatter_half


_scatter_lo = _make_scatter_half(0)
_scatter_hi = _make_scatter_half(RPH)


def kernel(hidden_states, extend_seq_lens, input_ids, W, b):
    tw0 = _token_weights(hidden_states, W, b, 0)
    o0 = _scatter_lo(input_ids, tw0)
    tw1 = _token_weights(hidden_states, W, b, 1)
    o1 = _scatter_hi(input_ids, tw1)
    out3 = jnp.concatenate([o0, o1], axis=1)
    return out3.transpose(1, 0, 2).reshape(B, VT * 128)[:, :V]


# speedup vs baseline: 1.1510x; 1.1510x over previous
import functools

import jax
import jax.numpy as jnp
from jax import lax
from jax.experimental import pallas as pl
from jax.experimental.pallas import tpu as pltpu
from jax.experimental.pallas import tpu_sc as plsc

B = 8
T = 8192
H = 1024
V = 100000

TOK = T // B
NW = 32
WPB = NW // B
NT = 196
VT = WPB * NT
VPW = NT * 128
NVREG = TOK // 16


def _token_weights_body(h_ref, w_ref, b_ref, o_ref):
    x = h_ref[...]
    w = w_ref[...]
    p3 = (x * w).reshape(o_ref.shape[0], 128, H)
    s = jnp.sum(p3, axis=2)
    o_ref[...] = jnp.maximum(s + b_ref[...], 0.0)


def _token_weights(hidden_states, W, b):
    blk = 2048
    return pl.pallas_call(
        _token_weights_body,
        grid=(T // blk,),
        in_specs=[
            pl.BlockSpec((blk, H), lambda i: (i, 0)),
            pl.BlockSpec((1, H), lambda i: (0, 0)),
            pl.BlockSpec((1, 1), lambda i: (0, 0)),
        ],
        out_specs=pl.BlockSpec((blk // 128, 128), lambda i: (i, 0)),
        out_shape=jax.ShapeDtypeStruct((T // 128, 128), jnp.float32),
    )(hidden_states, W, b.reshape(1, 1))


@functools.partial(
    pl.kernel,
    mesh=plsc.VectorSubcoreMesh(core_axis_name="c", subcore_axis_name="s"),
    out_type=jax.ShapeDtypeStruct((VT, B, 128), jnp.float32),
    compiler_params=pltpu.CompilerParams(
        needs_layout_passes=False, use_tc_tiling_on_sc=False),
    scratch_types=[
        pltpu.VMEM((TOK,), jnp.int32),
        pltpu.VMEM((TOK // 128, 128), jnp.float32),
        pltpu.VMEM((NT, 128), jnp.float32),
        pltpu.SemaphoreType.DMA,
        pltpu.SemaphoreType.DMA,
    ],
)
def _scatter_max(ids_hbm, tw_hbm, out_hbm, ids_v, tw_v, acc_v, sem_i, sem_w):
    wid = lax.axis_index("s") * 2 + lax.axis_index("c")
    row = wid // WPB
    q = wid % WPB
    v0 = q * VPW

    cp_i = pltpu.async_copy(ids_hbm.at[pl.ds(row * TOK, TOK)], ids_v, sem_i)
    cp_w = pltpu.async_copy(
        tw_hbm.at[pl.ds(row * (TOK // 128), TOK // 128), :], tw_v, sem_w)

    zeros16 = jnp.zeros((16,), jnp.float32)

    def zero_body(i, carry):
        for j in range(16):
            acc_v[i * 2 + j // 8, pl.ds((j % 8) * 16, 16)] = zeros16
        return carry

    lax.fori_loop(0, NT // 2, zero_body, 0)
    cp_i.wait()
    cp_w.wait()

    iota = jnp.arange(16, dtype=jnp.int32)

    def perm(x, idx):
        return jnp.take_along_axis(x, idx, axis=0)

    def one_vec(k):
        ids16 = ids_v[pl.ds(k * 16, 16)]
        w16 = tw_v[k // 8, pl.ds((k % 8) * 16, 16)]
        addr = ids16 - v0
        inr = (addr >= 0) & (addr < VPW)
        key = jnp.where(inr, addr, VPW)
        sk, sv = plsc.sort_key_val(key, w16)
        for s in (1, 2, 4, 8):
            up = jnp.minimum(iota + s, 15)
            k_sh = perm(sk, up)
            v_sh = perm(sv, up)
            sv = jnp.where(k_sh == sk, jnp.maximum(sv, v_sh), sv)
        k_dn = perm(sk, jnp.maximum(iota - 1, 0))
        head = (iota == 0) | (sk != k_dn)
        valid = head & (sk < VPW)
        skc = jnp.where(valid, sk, 0)
        t16 = skc // 128
        c16 = skc % 128
        cur = plsc.load_gather(acc_v, [t16, c16], mask=valid)
        plsc.store_scatter(acc_v, [t16, c16], jnp.maximum(cur, sv), mask=valid)

    def vec_body(i, carry):
        for j in range(2):
            one_vec(i * 2 + j)
        return carry

    lax.fori_loop(0, NVREG // 2, vec_body, 0)

    pltpu.sync_copy(acc_v, out_hbm.at[pl.ds(q * NT, NT), row, :])


def kernel(hidden_states, extend_seq_lens, input_ids, W, b):
    tw = _token_weights(hidden_states, W, b)
    out3 = _scatter_max(input_ids, tw)
    return out3.transpose(1, 0, 2).reshape(B, VT * 128)[:, :V]
